# baseline (speedup 1.0000x reference)
import jax
import jax.numpy as jnp
from jax.experimental import pallas as pl
from jax.experimental.pallas import tpu as pltpu

B = 262144
D = 64
H1 = 128
H2 = 64
BLK = 8192
M = BLK // 2


def _mlp_kernel(x_ref, w1_ref, b1_ref, w2_ref, b2_ref, w3_ref, b3_ref, out_ref):
    x = x_ref[...].astype(jnp.bfloat16)
    h = jnp.dot(x, w1_ref[...], preferred_element_type=jnp.float32)
    h = jnp.maximum(h + b1_ref[...], 0.0)
    h = jnp.dot(h.astype(jnp.bfloat16), w2_ref[...], preferred_element_type=jnp.float32)
    h = jnp.maximum(h + b2_ref[...], 0.0)
    t = h * w3_ref[...]
    sa = jnp.sum(t[:, :H2], axis=1) + b3_ref[0, 0]
    sb = jnp.sum(t[:, H2:], axis=1) + b3_ref[0, 0]
    a = BLK // 128
    out_ref[...] = jnp.concatenate(
        [sa.reshape(a, M // a), sb.reshape(a, M // a)], axis=1)


def kernel(x, W1, b1, W2, b2, W3, b3):
    w1 = W1.astype(jnp.bfloat16)
    w2 = W2.astype(jnp.bfloat16)
    z1 = jnp.zeros_like(w1)
    z2 = jnp.zeros_like(w2)
    w1d = jnp.block([[w1, z1], [z1, w1]])
    w2d = jnp.block([[w2, z2], [z2, w2]])
    b1d = jnp.tile(b1, 2).reshape(1, 2 * H1)
    b2d = jnp.tile(b2, 2).reshape(1, 2 * H2)
    w3d = jnp.tile(W3.reshape(H2), 2).reshape(1, 2 * H2)
    b3r = b3.reshape(1, 1)
    x2 = x.reshape(B // 2, 2 * D)
    grid = (B // BLK,)
    out2d = pl.pallas_call(
        _mlp_kernel,
        grid=grid,
        in_specs=[
            pl.BlockSpec((M, 2 * D), lambda i: (i, 0)),
            pl.BlockSpec((2 * D, 2 * H1), lambda i: (0, 0)),
            pl.BlockSpec((1, 2 * H1), lambda i: (0, 0)),
            pl.BlockSpec((2 * H1, 2 * H2), lambda i: (0, 0)),
            pl.BlockSpec((1, 2 * H2), lambda i: (0, 0)),
            pl.BlockSpec((1, 2 * H2), lambda i: (0, 0)),
            pl.BlockSpec((1, 1), lambda i: (0, 0)),
        ],
        out_specs=pl.BlockSpec((BLK // 128, 128), lambda i: (i, 0)),
        out_shape=jax.ShapeDtypeStruct((B // 128, 128), jnp.float32),
        compiler_params=pltpu.CompilerParams(
            dimension_semantics=("arbitrary",),
        ),
    )(x2, w1d, b1d, w2d, b2d, w3d, b3r)
    return out2d.reshape(B // 128, 2, 64).transpose(0, 2, 1).reshape(B, 1)
